# baseline (speedup 1.0000x reference)
import jax
import jax.numpy as jnp
from jax import lax
from jax.experimental import pallas as pl
from jax.experimental.pallas import tpu as pltpu
from jax.experimental.pallas import tpu_sc as plsc

_N = 10000
_E = 320000
_D = 128
_LN = 16
_NC = 2
_NS = 16
_NW = _NC * _NS
_CHUNK = 64
_CPT0 = 96
_CPT1 = 224
_BLK = 8
_EPAD = _NS * (_CPT0 + _CPT1) * _CHUNK
_RPW = 640
_EPS = 1e-5


def _lap_body(src_hbm, dst_hbm, ev_hbm, u_hbm, zero_hbm, out_hbm,
              src_v, dst_v, ev_v, rows_v, accum,
              gsem0, gsem1, gsem2, gsem3, ssem0, ssem1, ssem2, ssem3, rsem):
    c = lax.axis_index("c")
    s = lax.axis_index("s")
    cpt = jnp.where(c == 0, _CPT0, _CPT1)
    base = jnp.where(c == 0, s * _CPT0, _NS * _CPT0 + s * _CPT1)

    def refill_copies(blk, par):
        hb = pl.ds(base + blk * _BLK, _BLK)
        return ((src_hbm.at[hb], src_v.at[par]),
                (dst_hbm.at[hb], dst_v.at[par]),
                (ev_hbm.at[hb], ev_v.at[par]))

    for a, b in refill_copies(0, 0):
        pltpu.sync_copy(a, b)

    row0 = s * _RPW

    @pl.when(s < _NS - 1)
    def _():
        pltpu.sync_copy(zero_hbm.at[pl.ds(row0, _RPW)],
                        accum.at[pl.ds(row0, _RPW)])

    @pl.when(s == _NS - 1)
    def _():
        pltpu.sync_copy(zero_hbm.at[pl.ds(row0, _N - (_NS - 1) * _RPW)],
                        accum.at[pl.ds(row0, _N - (_NS - 1) * _RPW)])

    plsc.subcore_barrier()

    def scale(b, par, jm):
        def group(g, c2):
            ew = ev_v[par, jm, pl.ds(g * _LN, _LN)]
            for k in range(_LN):
                i = g * _LN + k
                eb = jnp.full((_LN,), ew[k], jnp.float32)
                for t in range(_D // _LN):
                    sl = pl.ds(t * _LN, _LN)
                    rows_v[b, i, sl] = rows_v[b, i, sl] * eb
            return c2

        lax.fori_loop(0, _CHUNK // _LN, group, 0)

    gsems = (gsem0, gsem1, gsem2, gsem3)
    ssems = (ssem0, ssem1, ssem2, ssem3)

    @pl.when(_BLK < cpt)
    def _():
        for a, b in refill_copies(1, 1):
            pltpu.async_copy(a, b, rsem)

    for q0 in range(3):
        @pl.when(q0 < cpt)
        def _(q0=q0):
            pltpu.async_copy(u_hbm.at[src_v.at[0, q0]], rows_v.at[q0],
                             gsems[q0])

    def chunk_step(j, q):
        jm = lax.rem(j, _BLK)
        blk = lax.div(j, _BLK)
        par = lax.rem(blk, 2)
        pltpu.make_async_copy(u_hbm.at[src_v.at[par, jm]], rows_v.at[q],
                              gsems[q]).wait()
        scale(q, par, jm)
        pltpu.async_copy(rows_v.at[q], accum.at[dst_v.at[par, jm]],
                         ssems[q], add=True)
        qp = (q + 3) % 4

        @pl.when(j > 0)
        def _():
            pltpu.make_async_copy(rows_v.at[qp], accum.at[dst_v.at[par, jm]],
                                  ssems[qp]).wait()

        @pl.when((jm == 0) & ((blk + 1) * _BLK < cpt))
        def _():
            for a, b in refill_copies(blk + 1, 1 - par):
                pltpu.async_copy(a, b, rsem)

        @pl.when((jm == 4) & ((blk + 1) * _BLK < cpt))
        def _():
            for a, b in refill_copies(blk + 1, 1 - par):
                pltpu.make_async_copy(a, b, rsem).wait()

        @pl.when(j + 3 < cpt)
        def _():
            j3 = j + 3
            jm3 = lax.rem(j3, _BLK)
            par3 = lax.rem(lax.div(j3, _BLK), 2)
            pltpu.async_copy(u_hbm.at[src_v.at[par3, jm3]], rows_v.at[qp],
                             gsems[qp])

    def quad(h, carry):
        for q in range(4):
            chunk_step(4 * h + q, q)
        return carry

    lax.fori_loop(0, cpt // 4, quad, 0)

    @pl.when(cpt > 0)
    def _():
        pltpu.make_async_copy(rows_v.at[3], accum.at[dst_v.at[0, 0]],
                              ssems[3]).wait()
    plsc.subcore_barrier()

    @pl.when(s < _NS - 1)
    def _():
        pltpu.sync_copy(accum.at[pl.ds(row0, _RPW)],
                        out_hbm.at[c, pl.ds(row0, _RPW)])

    @pl.when(s == _NS - 1)
    def _():
        pltpu.sync_copy(accum.at[pl.ds(row0, _N - (_NS - 1) * _RPW)],
                        out_hbm.at[c, pl.ds(row0, _N - (_NS - 1) * _RPW)])


_LAP_CACHE = []


def _get_lap():
    if not _LAP_CACHE:
        _LAP_CACHE.append(pl.kernel(
            _lap_body,
            out_type=jax.ShapeDtypeStruct((_NC, _N, _D), jnp.float32),
            mesh=plsc.VectorSubcoreMesh(core_axis_name="c",
                                        subcore_axis_name="s"),
            scratch_types=[
                pltpu.VMEM((2, _BLK, _CHUNK), jnp.int32),
                pltpu.VMEM((2, _BLK, _CHUNK), jnp.int32),
                pltpu.VMEM((2, _BLK, _CHUNK), jnp.float32),
                pltpu.VMEM((4, _CHUNK, _D), jnp.float32),
                pltpu.VMEM_SHARED((_N, _D), jnp.float32),
            ] + [pltpu.SemaphoreType.DMA] * 9,
        ))
    return _LAP_CACHE[0]


def _elu(x):
    return jnp.where(x > 0, x, jnp.exp(x) - 1.0)


def _stats(u):
    m = jnp.mean(u, axis=0)
    v = jnp.mean((u - m) ** 2, axis=0)
    return m, v


def _bn(u, m, v, g, b):
    return (u - m) * lax.rsqrt(v + _EPS) * g + b


def _gc_even(u, op, g, bt, W, b):
    mu, vu = _stats(u)
    mo, vo = _stats(op)
    un = _bn(u, mu, vu, g[:_D], bt[:_D])
    on = _bn(op, mo, vo, g[_D:], bt[_D:])
    return (jnp.dot(un, W[:_D], preferred_element_type=jnp.float32)
            + jnp.dot(on, W[_D:], preferred_element_type=jnp.float32)
            + b[None, :])


def _gc_odd(u, g, bt, W, b):
    mu, vu = _stats(u)
    un = _bn(u, mu, vu, g[:_D], bt[:_D])
    const = jnp.dot(bt[_D:][None, :], W[_D:],
                    preferred_element_type=jnp.float32)
    return (jnp.dot(un, W[:_D], preferred_element_type=jnp.float32)
            + const + b[None, :])


def _stage_in_body(inp_ref, W_ref, b_ref, x_ref, u_ref):
    x = (jnp.dot(inp_ref[...], W_ref[...], preferred_element_type=jnp.float32)
         + b_ref[...][None, :])
    x_ref[...] = x
    u_ref[...] = _elu(x)


def _stage_even_a_body(u_ref, p_ref, g_ref, bt_ref, W_ref, b_ref, u2_ref):
    op = p_ref[0] + p_ref[1]
    y = _gc_even(u_ref[...], op, g_ref[...], bt_ref[...], W_ref[...],
                 b_ref[...])
    u2_ref[...] = _elu(y)


def _stage_mid_body(u2_ref, p_ref, g_b, bt_b, W_b, b_b, xp_ref,
                    g0, bt0, W0, b0, g1, bt1, W1, b1, x_ref, un_ref):
    op = p_ref[0] + p_ref[1]
    z = _gc_even(u2_ref[...], op, g_b[...], bt_b[...], W_b[...], b_b[...])
    x1 = z + xp_ref[...]
    u = _elu(x1)
    y = _gc_odd(u, g0[...], bt0[...], W0[...], b0[...])
    z2 = _gc_odd(_elu(y), g1[...], bt1[...], W1[...], b1[...])
    x2 = z2 + x1
    x_ref[...] = x2
    un_ref[...] = _elu(x2)


def _stage_fin_body(u2_ref, p_ref, g_b, bt_b, W_b, b_b, xp_ref,
                    g0, bt0, W0, b0, g1, bt1, W1, b1,
                    cg, cb, cW, cbb, tiled_ref, out_ref):
    op = p_ref[0] + p_ref[1]
    z = _gc_even(u2_ref[...], op, g_b[...], bt_b[...], W_b[...], b_b[...])
    x1 = z + xp_ref[...]
    u = _elu(x1)
    y = _gc_odd(u, g0[...], bt0[...], W0[...], b0[...])
    z2 = _gc_odd(_elu(y), g1[...], bt1[...], W1[...], b1[...])
    x2 = z2 + x1
    uf = _elu(x2)
    m, v = _stats(uf)
    out = (jnp.dot(_bn(uf, m, v, cg[...], cb[...]), cW[...],
                   preferred_element_type=jnp.float32) + cbb[...][None, :])
    out_ref[...] = out + tiled_ref[...]


def kernel(inputs, mask, edge_index, edge_values, W_in, b_in,
           fc0_gamma, fc0_beta, fc0_W, fc0_b,
           fc1_gamma, fc1_beta, fc1_W, fc1_b,
           conv2_gamma, conv2_beta, conv2_W, conv2_b):
    del mask
    f32 = jnp.float32
    inp3 = inputs[0]
    pad = _EPAD - _E
    zpad_i = jnp.zeros((pad,), jnp.int32)
    src = jnp.concatenate([edge_index[0].astype(jnp.int32), zpad_i]
                          ).reshape(-1, _CHUNK)
    dst = jnp.concatenate([edge_index[1].astype(jnp.int32), zpad_i]
                          ).reshape(-1, _CHUNK)
    ev = jnp.concatenate([edge_values.astype(f32), jnp.zeros((pad,), f32)]
                         ).reshape(-1, _CHUNK)
    zeros = jnp.zeros((_N, _D), f32)
    tiled = jnp.tile(inp3[:, -3:], (1, 40))

    sd = lambda shape: jax.ShapeDtypeStruct(shape, f32)

    x, u = pl.pallas_call(
        _stage_in_body, out_shape=[sd((_N, _D)), sd((_N, _D))])(
            inp3, W_in, b_in)

    p = _get_lap()(src, dst, ev, u, zeros)
    u2 = pl.pallas_call(_stage_even_a_body, out_shape=sd((_N, _D)))(
        u, p, fc0_gamma[0], fc0_beta[0], fc0_W[0], fc0_b[0])

    p = _get_lap()(src, dst, ev, u2, zeros)
    x, u = pl.pallas_call(
        _stage_mid_body, out_shape=[sd((_N, _D)), sd((_N, _D))])(
            u2, p, fc1_gamma[0], fc1_beta[0], fc1_W[0], fc1_b[0], x,
            fc0_gamma[1], fc0_beta[1], fc0_W[1], fc0_b[1],
            fc1_gamma[1], fc1_beta[1], fc1_W[1], fc1_b[1])

    p = _get_lap()(src, dst, ev, u, zeros)
    u2 = pl.pallas_call(_stage_even_a_body, out_shape=sd((_N, _D)))(
        u, p, fc0_gamma[2], fc0_beta[2], fc0_W[2], fc0_b[2])

    p = _get_lap()(src, dst, ev, u2, zeros)
    out = pl.pallas_call(_stage_fin_body, out_shape=sd((_N, 120)))(
        u2, p, fc1_gamma[2], fc1_beta[2], fc1_W[2], fc1_b[2], x,
        fc0_gamma[3], fc0_beta[3], fc0_W[3], fc0_b[3],
        fc1_gamma[3], fc1_beta[3], fc1_W[3], fc1_b[3],
        conv2_gamma, conv2_beta, conv2_W, conv2_b, tiled)
    return out[None]


# speedup vs baseline: 1.0677x; 1.0677x over previous
import jax
import jax.numpy as jnp
from jax import lax
from jax.experimental import pallas as pl
from jax.experimental.pallas import tpu as pltpu
from jax.experimental.pallas import tpu_sc as plsc

_N = 10000
_E = 320000
_D = 128
_LN = 16
_NC = 2
_NS = 16
_NW = _NC * _NS
_CHUNK = 64
_CPT0 = 224
_CPT1 = 96
_BLK = 8
_EPAD = _NS * (_CPT0 + _CPT1) * _CHUNK
_RPW = 640
_EPS = 1e-5


def _lap_body(src_hbm, dst_hbm, ev_hbm, u_hbm, zero_hbm, out_hbm,
              src_v, dst_v, ev_v, rows_v, accum,
              gsem0, gsem1, gsem2, gsem3, ssem0, ssem1, ssem2, ssem3, rsem):
    c = lax.axis_index("c")
    s = lax.axis_index("s")
    cpt = jnp.where(c == 0, _CPT0, _CPT1)
    base = jnp.where(c == 0, s * _CPT0, _NS * _CPT0 + s * _CPT1)

    def refill_copies(blk, par):
        hb = pl.ds(base + blk * _BLK, _BLK)
        return ((src_hbm.at[hb], src_v.at[par]),
                (dst_hbm.at[hb], dst_v.at[par]),
                (ev_hbm.at[hb], ev_v.at[par]))

    for a, b in refill_copies(0, 0):
        pltpu.sync_copy(a, b)

    row0 = s * _RPW

    @pl.when(s < _NS - 1)
    def _():
        pltpu.sync_copy(zero_hbm.at[pl.ds(row0, _RPW)],
                        accum.at[pl.ds(row0, _RPW)])

    @pl.when(s == _NS - 1)
    def _():
        pltpu.sync_copy(zero_hbm.at[pl.ds(row0, _N - (_NS - 1) * _RPW)],
                        accum.at[pl.ds(row0, _N - (_NS - 1) * _RPW)])

    plsc.subcore_barrier()

    def scale(b, par, jm):
        def group(g, c2):
            ew = ev_v[par, jm, pl.ds(g * _LN, _LN)]
            for k in range(_LN):
                i = g * _LN + k
                eb = jnp.full((_LN,), ew[k], jnp.float32)
                for t in range(_D // _LN):
                    sl = pl.ds(t * _LN, _LN)
                    rows_v[b, i, sl] = rows_v[b, i, sl] * eb
            return c2

        lax.fori_loop(0, _CHUNK // _LN, group, 0)

    gsems = (gsem0, gsem1, gsem2, gsem3)
    ssems = (ssem0, ssem1, ssem2, ssem3)

    @pl.when(_BLK < cpt)
    def _():
        for a, b in refill_copies(1, 1):
            pltpu.async_copy(a, b, rsem)

    for q0 in range(3):
        @pl.when(q0 < cpt)
        def _(q0=q0):
            pltpu.async_copy(u_hbm.at[src_v.at[0, q0]], rows_v.at[q0],
                             gsems[q0])

    def chunk_step(j, q):
        jm = lax.rem(j, _BLK)
        blk = lax.div(j, _BLK)
        par = lax.rem(blk, 2)
        pltpu.make_async_copy(u_hbm.at[src_v.at[par, jm]], rows_v.at[q],
                              gsems[q]).wait()
        scale(q, par, jm)
        pltpu.async_copy(rows_v.at[q], accum.at[dst_v.at[par, jm]],
                         ssems[q], add=True)
        qp = (q + 3) % 4

        @pl.when(j > 0)
        def _():
            pltpu.make_async_copy(rows_v.at[qp], accum.at[dst_v.at[par, jm]],
                                  ssems[qp]).wait()

        @pl.when((jm == 0) & ((blk + 1) * _BLK < cpt))
        def _():
            for a, b in refill_copies(blk + 1, 1 - par):
                pltpu.async_copy(a, b, rsem)

        @pl.when((jm == 4) & ((blk + 1) * _BLK < cpt))
        def _():
            for a, b in refill_copies(blk + 1, 1 - par):
                pltpu.make_async_copy(a, b, rsem).wait()

        @pl.when(j + 3 < cpt)
        def _():
            j3 = j + 3
            jm3 = lax.rem(j3, _BLK)
            par3 = lax.rem(lax.div(j3, _BLK), 2)
            pltpu.async_copy(u_hbm.at[src_v.at[par3, jm3]], rows_v.at[qp],
                             gsems[qp])

    def quad(h, carry):
        for q in range(4):
            chunk_step(4 * h + q, q)
        return carry

    lax.fori_loop(0, cpt // 4, quad, 0)

    @pl.when(cpt > 0)
    def _():
        pltpu.make_async_copy(rows_v.at[3], accum.at[dst_v.at[0, 0]],
                              ssems[3]).wait()
    plsc.subcore_barrier()

    @pl.when(s < _NS - 1)
    def _():
        pltpu.sync_copy(accum.at[pl.ds(row0, _RPW)],
                        out_hbm.at[c, pl.ds(row0, _RPW)])

    @pl.when(s == _NS - 1)
    def _():
        pltpu.sync_copy(accum.at[pl.ds(row0, _N - (_NS - 1) * _RPW)],
                        out_hbm.at[c, pl.ds(row0, _N - (_NS - 1) * _RPW)])


_LAP_CACHE = []


def _get_lap():
    if not _LAP_CACHE:
        _LAP_CACHE.append(pl.kernel(
            _lap_body,
            out_type=jax.ShapeDtypeStruct((_NC, _N, _D), jnp.float32),
            mesh=plsc.VectorSubcoreMesh(core_axis_name="c",
                                        subcore_axis_name="s"),
            scratch_types=[
                pltpu.VMEM((2, _BLK, _CHUNK), jnp.int32),
                pltpu.VMEM((2, _BLK, _CHUNK), jnp.int32),
                pltpu.VMEM((2, _BLK, _CHUNK), jnp.float32),
                pltpu.VMEM((4, _CHUNK, _D), jnp.float32),
                pltpu.VMEM_SHARED((_N, _D), jnp.float32),
            ] + [pltpu.SemaphoreType.DMA] * 9,
        ))
    return _LAP_CACHE[0]


def _elu(x):
    return jnp.where(x > 0, x, jnp.exp(x) - 1.0)


def _stats(u):
    m = jnp.mean(u, axis=0)
    v = jnp.mean((u - m) ** 2, axis=0)
    return m, v


def _bn(u, m, v, g, b):
    return (u - m) * lax.rsqrt(v + _EPS) * g + b


def _gc_even(u, op, g, bt, W, b):
    mu, vu = _stats(u)
    mo, vo = _stats(op)
    un = _bn(u, mu, vu, g[:_D], bt[:_D])
    on = _bn(op, mo, vo, g[_D:], bt[_D:])
    return (jnp.dot(un, W[:_D], preferred_element_type=jnp.float32)
            + jnp.dot(on, W[_D:], preferred_element_type=jnp.float32)
            + b[None, :])


def _gc_odd(u, g, bt, W, b):
    mu, vu = _stats(u)
    un = _bn(u, mu, vu, g[:_D], bt[:_D])
    const = jnp.dot(bt[_D:][None, :], W[_D:],
                    preferred_element_type=jnp.float32)
    return (jnp.dot(un, W[:_D], preferred_element_type=jnp.float32)
            + const + b[None, :])


def _stage_in_body(inp_ref, W_ref, b_ref, x_ref, u_ref):
    x = (jnp.dot(inp_ref[...], W_ref[...], preferred_element_type=jnp.float32)
         + b_ref[...][None, :])
    x_ref[...] = x
    u_ref[...] = _elu(x)


def _stage_even_a_body(u_ref, p_ref, g_ref, bt_ref, W_ref, b_ref, u2_ref):
    op = p_ref[0] + p_ref[1]
    y = _gc_even(u_ref[...], op, g_ref[...], bt_ref[...], W_ref[...],
                 b_ref[...])
    u2_ref[...] = _elu(y)


def _stage_mid_body(u2_ref, p_ref, g_b, bt_b, W_b, b_b, xp_ref,
                    g0, bt0, W0, b0, g1, bt1, W1, b1, x_ref, un_ref):
    op = p_ref[0] + p_ref[1]
    z = _gc_even(u2_ref[...], op, g_b[...], bt_b[...], W_b[...], b_b[...])
    x1 = z + xp_ref[...]
    u = _elu(x1)
    y = _gc_odd(u, g0[...], bt0[...], W0[...], b0[...])
    z2 = _gc_odd(_elu(y), g1[...], bt1[...], W1[...], b1[...])
    x2 = z2 + x1
    x_ref[...] = x2
    un_ref[...] = _elu(x2)


def _stage_fin_body(u2_ref, p_ref, g_b, bt_b, W_b, b_b, xp_ref,
                    g0, bt0, W0, b0, g1, bt1, W1, b1,
                    cg, cb, cW, cbb, tiled_ref, out_ref):
    op = p_ref[0] + p_ref[1]
    z = _gc_even(u2_ref[...], op, g_b[...], bt_b[...], W_b[...], b_b[...])
    x1 = z + xp_ref[...]
    u = _elu(x1)
    y = _gc_odd(u, g0[...], bt0[...], W0[...], b0[...])
    z2 = _gc_odd(_elu(y), g1[...], bt1[...], W1[...], b1[...])
    x2 = z2 + x1
    uf = _elu(x2)
    m, v = _stats(uf)
    out = (jnp.dot(_bn(uf, m, v, cg[...], cb[...]), cW[...],
                   preferred_element_type=jnp.float32) + cbb[...][None, :])
    out_ref[...] = out + tiled_ref[...]


def kernel(inputs, mask, edge_index, edge_values, W_in, b_in,
           fc0_gamma, fc0_beta, fc0_W, fc0_b,
           fc1_gamma, fc1_beta, fc1_W, fc1_b,
           conv2_gamma, conv2_beta, conv2_W, conv2_b):
    del mask
    f32 = jnp.float32
    inp3 = inputs[0]
    pad = _EPAD - _E
    zpad_i = jnp.zeros((pad,), jnp.int32)
    src = jnp.concatenate([edge_index[0].astype(jnp.int32), zpad_i]
                          ).reshape(-1, _CHUNK)
    dst = jnp.concatenate([edge_index[1].astype(jnp.int32), zpad_i]
                          ).reshape(-1, _CHUNK)
    ev = jnp.concatenate([edge_values.astype(f32), jnp.zeros((pad,), f32)]
                         ).reshape(-1, _CHUNK)
    zeros = jnp.zeros((_N, _D), f32)
    tiled = jnp.tile(inp3[:, -3:], (1, 40))

    sd = lambda shape: jax.ShapeDtypeStruct(shape, f32)

    x, u = pl.pallas_call(
        _stage_in_body, out_shape=[sd((_N, _D)), sd((_N, _D))])(
            inp3, W_in, b_in)

    p = _get_lap()(src, dst, ev, u, zeros)
    u2 = pl.pallas_call(_stage_even_a_body, out_shape=sd((_N, _D)))(
        u, p, fc0_gamma[0], fc0_beta[0], fc0_W[0], fc0_b[0])

    p = _get_lap()(src, dst, ev, u2, zeros)
    x, u = pl.pallas_call(
        _stage_mid_body, out_shape=[sd((_N, _D)), sd((_N, _D))])(
            u2, p, fc1_gamma[0], fc1_beta[0], fc1_W[0], fc1_b[0], x,
            fc0_gamma[1], fc0_beta[1], fc0_W[1], fc0_b[1],
            fc1_gamma[1], fc1_beta[1], fc1_W[1], fc1_b[1])

    p = _get_lap()(src, dst, ev, u, zeros)
    u2 = pl.pallas_call(_stage_even_a_body, out_shape=sd((_N, _D)))(
        u, p, fc0_gamma[2], fc0_beta[2], fc0_W[2], fc0_b[2])

    p = _get_lap()(src, dst, ev, u2, zeros)
    out = pl.pallas_call(_stage_fin_body, out_shape=sd((_N, 120)))(
        u2, p, fc1_gamma[2], fc1_beta[2], fc1_W[2], fc1_b[2], x,
        fc0_gamma[3], fc0_beta[3], fc0_W[3], fc0_b[3],
        fc1_gamma[3], fc1_beta[3], fc1_W[3], fc1_b[3],
        conv2_gamma, conv2_beta, conv2_W, conv2_b, tiled)
    return out[None]


# speedup vs baseline: 1.0695x; 1.0016x over previous
import jax
import jax.numpy as jnp
from jax import lax
from jax.experimental import pallas as pl
from jax.experimental.pallas import tpu as pltpu
from jax.experimental.pallas import tpu_sc as plsc

_N = 10000
_E = 320000
_D = 128
_LN = 16
_NC = 2
_NS = 16
_NW = _NC * _NS
_CHUNK = 64
_CPT0 = 224
_CPT1 = 96
_BLK = 8
_EPAD = _NS * (_CPT0 + _CPT1) * _CHUNK
_RPW = 640
_EPS = 1e-5


def _lap_body(src_hbm, dst_hbm, ev_hbm, u_hbm, zero_hbm, out_hbm,
              src_v, dst_v, ev_v, rows_v, accum,
              gsem0, gsem1, gsem2, gsem3, ssem0, ssem1, ssem2, ssem3, rsem):
    c = lax.axis_index("c")
    s = lax.axis_index("s")
    cpt = jnp.where(c == 0, _CPT0, _CPT1)
    base = jnp.where(c == 0, s * _CPT0, _NS * _CPT0 + s * _CPT1)

    def refill_copies(blk, par):
        hb = pl.ds(base + blk * _BLK, _BLK)
        return ((src_hbm.at[hb], src_v.at[par]),
                (dst_hbm.at[hb], dst_v.at[par]),
                (ev_hbm.at[hb], ev_v.at[par]))

    for a, b in refill_copies(0, 0):
        pltpu.sync_copy(a, b)

    row0 = s * _RPW

    @pl.when(s < _NS - 1)
    def _():
        pltpu.sync_copy(zero_hbm.at[pl.ds(row0, _RPW)],
                        accum.at[pl.ds(row0, _RPW)])

    @pl.when(s == _NS - 1)
    def _():
        pltpu.sync_copy(zero_hbm.at[pl.ds(row0, _N - (_NS - 1) * _RPW)],
                        accum.at[pl.ds(row0, _N - (_NS - 1) * _RPW)])

    plsc.subcore_barrier()

    def scale(b, par, jm):
        def group(g, c2):
            ew = ev_v[par, jm, pl.ds(g * _LN, _LN)]
            for k in range(_LN):
                i = g * _LN + k
                eb = jnp.full((_LN,), ew[k], jnp.float32)
                for t in range(_D // _LN):
                    sl = pl.ds(t * _LN, _LN)
                    rows_v[b, i, sl] = rows_v[b, i, sl] * eb
            return c2

        lax.fori_loop(0, _CHUNK // _LN, group, 0)

    gsems = (gsem0, gsem1, gsem2, gsem3)
    ssems = (ssem0, ssem1, ssem2, ssem3)

    @pl.when(_BLK < cpt)
    def _():
        for a, b in refill_copies(1, 1):
            pltpu.async_copy(a, b, rsem)

    for q0 in range(3):
        @pl.when(q0 < cpt)
        def _(q0=q0):
            pltpu.async_copy(u_hbm.at[src_v.at[0, q0]], rows_v.at[q0],
                             gsems[q0])

    def chunk_step(j, q):
        jm = lax.rem(j, _BLK)
        blk = lax.div(j, _BLK)
        par = lax.rem(blk, 2)
        pltpu.make_async_copy(u_hbm.at[src_v.at[par, jm]], rows_v.at[q],
                              gsems[q]).wait()
        scale(q, par, jm)
        pltpu.async_copy(rows_v.at[q], accum.at[dst_v.at[par, jm]],
                         ssems[q], add=True)
        qp = (q + 3) % 4

        @pl.when(j > 0)
        def _():
            pltpu.make_async_copy(rows_v.at[qp], accum.at[dst_v.at[par, jm]],
                                  ssems[qp]).wait()

        @pl.when((jm == 0) & ((blk + 1) * _BLK < cpt))
        def _():
            for a, b in refill_copies(blk + 1, 1 - par):
                pltpu.async_copy(a, b, rsem)

        @pl.when((jm == 4) & ((blk + 1) * _BLK < cpt))
        def _():
            for a, b in refill_copies(blk + 1, 1 - par):
                pltpu.make_async_copy(a, b, rsem).wait()

        @pl.when(j + 3 < cpt)
        def _():
            j3 = j + 3
            jm3 = lax.rem(j3, _BLK)
            par3 = lax.rem(lax.div(j3, _BLK), 2)
            pltpu.async_copy(u_hbm.at[src_v.at[par3, jm3]], rows_v.at[qp],
                             gsems[qp], priority=1)

    def quad(h, carry):
        for q in range(4):
            chunk_step(4 * h + q, q)
        return carry

    lax.fori_loop(0, cpt // 4, quad, 0)

    @pl.when(cpt > 0)
    def _():
        pltpu.make_async_copy(rows_v.at[3], accum.at[dst_v.at[0, 0]],
                              ssems[3]).wait()
    plsc.subcore_barrier()

    @pl.when(s < _NS - 1)
    def _():
        pltpu.sync_copy(accum.at[pl.ds(row0, _RPW)],
                        out_hbm.at[c, pl.ds(row0, _RPW)])

    @pl.when(s == _NS - 1)
    def _():
        pltpu.sync_copy(accum.at[pl.ds(row0, _N - (_NS - 1) * _RPW)],
                        out_hbm.at[c, pl.ds(row0, _N - (_NS - 1) * _RPW)])


_LAP_CACHE = []


def _get_lap():
    if not _LAP_CACHE:
        _LAP_CACHE.append(pl.kernel(
            _lap_body,
            out_type=jax.ShapeDtypeStruct((_NC, _N, _D), jnp.float32),
            mesh=plsc.VectorSubcoreMesh(core_axis_name="c",
                                        subcore_axis_name="s"),
            scratch_types=[
                pltpu.VMEM((2, _BLK, _CHUNK), jnp.int32),
                pltpu.VMEM((2, _BLK, _CHUNK), jnp.int32),
                pltpu.VMEM((2, _BLK, _CHUNK), jnp.float32),
                pltpu.VMEM((4, _CHUNK, _D), jnp.float32),
                pltpu.VMEM_SHARED((_N, _D), jnp.float32),
            ] + [pltpu.SemaphoreType.DMA] * 9,
        ))
    return _LAP_CACHE[0]


def _elu(x):
    return jnp.where(x > 0, x, jnp.exp(x) - 1.0)


def _stats(u):
    m = jnp.mean(u, axis=0)
    v = jnp.mean((u - m) ** 2, axis=0)
    return m, v


def _bn(u, m, v, g, b):
    return (u - m) * lax.rsqrt(v + _EPS) * g + b


def _gc_even(u, op, g, bt, W, b):
    mu, vu = _stats(u)
    mo, vo = _stats(op)
    un = _bn(u, mu, vu, g[:_D], bt[:_D])
    on = _bn(op, mo, vo, g[_D:], bt[_D:])
    return (jnp.dot(un, W[:_D], preferred_element_type=jnp.float32)
            + jnp.dot(on, W[_D:], preferred_element_type=jnp.float32)
            + b[None, :])


def _gc_odd(u, g, bt, W, b):
    mu, vu = _stats(u)
    un = _bn(u, mu, vu, g[:_D], bt[:_D])
    const = jnp.dot(bt[_D:][None, :], W[_D:],
                    preferred_element_type=jnp.float32)
    return (jnp.dot(un, W[:_D], preferred_element_type=jnp.float32)
            + const + b[None, :])


def _stage_in_body(inp_ref, W_ref, b_ref, x_ref, u_ref):
    x = (jnp.dot(inp_ref[...], W_ref[...], preferred_element_type=jnp.float32)
         + b_ref[...][None, :])
    x_ref[...] = x
    u_ref[...] = _elu(x)


def _stage_even_a_body(u_ref, p_ref, g_ref, bt_ref, W_ref, b_ref, u2_ref):
    op = p_ref[0] + p_ref[1]
    y = _gc_even(u_ref[...], op, g_ref[...], bt_ref[...], W_ref[...],
                 b_ref[...])
    u2_ref[...] = _elu(y)


def _stage_mid_body(u2_ref, p_ref, g_b, bt_b, W_b, b_b, xp_ref,
                    g0, bt0, W0, b0, g1, bt1, W1, b1, x_ref, un_ref):
    op = p_ref[0] + p_ref[1]
    z = _gc_even(u2_ref[...], op, g_b[...], bt_b[...], W_b[...], b_b[...])
    x1 = z + xp_ref[...]
    u = _elu(x1)
    y = _gc_odd(u, g0[...], bt0[...], W0[...], b0[...])
    z2 = _gc_odd(_elu(y), g1[...], bt1[...], W1[...], b1[...])
    x2 = z2 + x1
    x_ref[...] = x2
    un_ref[...] = _elu(x2)


def _stage_fin_body(u2_ref, p_ref, g_b, bt_b, W_b, b_b, xp_ref,
                    g0, bt0, W0, b0, g1, bt1, W1, b1,
                    cg, cb, cW, cbb, tiled_ref, out_ref):
    op = p_ref[0] + p_ref[1]
    z = _gc_even(u2_ref[...], op, g_b[...], bt_b[...], W_b[...], b_b[...])
    x1 = z + xp_ref[...]
    u = _elu(x1)
    y = _gc_odd(u, g0[...], bt0[...], W0[...], b0[...])
    z2 = _gc_odd(_elu(y), g1[...], bt1[...], W1[...], b1[...])
    x2 = z2 + x1
    uf = _elu(x2)
    m, v = _stats(uf)
    out = (jnp.dot(_bn(uf, m, v, cg[...], cb[...]), cW[...],
                   preferred_element_type=jnp.float32) + cbb[...][None, :])
    out_ref[...] = out + tiled_ref[...]


def kernel(inputs, mask, edge_index, edge_values, W_in, b_in,
           fc0_gamma, fc0_beta, fc0_W, fc0_b,
           fc1_gamma, fc1_beta, fc1_W, fc1_b,
           conv2_gamma, conv2_beta, conv2_W, conv2_b):
    del mask
    f32 = jnp.float32
    inp3 = inputs[0]
    pad = _EPAD - _E
    zpad_i = jnp.zeros((pad,), jnp.int32)
    src = jnp.concatenate([edge_index[0].astype(jnp.int32), zpad_i]
                          ).reshape(-1, _CHUNK)
    dst = jnp.concatenate([edge_index[1].astype(jnp.int32), zpad_i]
                          ).reshape(-1, _CHUNK)
    ev = jnp.concatenate([edge_values.astype(f32), jnp.zeros((pad,), f32)]
                         ).reshape(-1, _CHUNK)
    zeros = jnp.zeros((_N, _D), f32)
    tiled = jnp.tile(inp3[:, -3:], (1, 40))

    sd = lambda shape: jax.ShapeDtypeStruct(shape, f32)

    x, u = pl.pallas_call(
        _stage_in_body, out_shape=[sd((_N, _D)), sd((_N, _D))])(
            inp3, W_in, b_in)

    p = _get_lap()(src, dst, ev, u, zeros)
    u2 = pl.pallas_call(_stage_even_a_body, out_shape=sd((_N, _D)))(
        u, p, fc0_gamma[0], fc0_beta[0], fc0_W[0], fc0_b[0])

    p = _get_lap()(src, dst, ev, u2, zeros)
    x, u = pl.pallas_call(
        _stage_mid_body, out_shape=[sd((_N, _D)), sd((_N, _D))])(
            u2, p, fc1_gamma[0], fc1_beta[0], fc1_W[0], fc1_b[0], x,
            fc0_gamma[1], fc0_beta[1], fc0_W[1], fc0_b[1],
            fc1_gamma[1], fc1_beta[1], fc1_W[1], fc1_b[1])

    p = _get_lap()(src, dst, ev, u, zeros)
    u2 = pl.pallas_call(_stage_even_a_body, out_shape=sd((_N, _D)))(
        u, p, fc0_gamma[2], fc0_beta[2], fc0_W[2], fc0_b[2])

    p = _get_lap()(src, dst, ev, u2, zeros)
    out = pl.pallas_call(_stage_fin_body, out_shape=sd((_N, 120)))(
        u2, p, fc1_gamma[2], fc1_beta[2], fc1_W[2], fc1_b[2], x,
        fc0_gamma[3], fc0_beta[3], fc0_W[3], fc0_b[3],
        fc1_gamma[3], fc1_beta[3], fc1_W[3], fc1_b[3],
        conv2_gamma, conv2_beta, conv2_W, conv2_b, tiled)
    return out[None]
